# Initial kernel scaffold; baseline (speedup 1.0000x reference)
import jax
import jax.numpy as jnp
from jax.experimental import pallas as pl


def kernel(nf, ef, edge_index, etype, ntype, We, be, Wn, bn):
    raise NotImplementedError("write your pallas kernel here")


# speedup vs baseline: 3.7664x; 3.7664x over previous
import functools

import jax
import jax.numpy as jnp
from jax import lax
from jax.experimental import pallas as pl
from jax.experimental.pallas import tpu as pltpu
from jax.experimental.pallas import tpu_sc as plsc

N = 10000
E = 320000
DF = 128
DE = 16
TE = 3
TN = 2

NW = 32
CHUNK = 128
NCHUNK = 80
PW = CHUNK * NCHUNK
EP = NW * PW
ROWS = EP * DE // 128
ROWS_REAL = E * DE // 128

_HIGH = lax.Precision.HIGHEST


def _dot(a, b):
    return lax.dot_general(a, b, (((1,), (0,)), ((), ())),
                           precision=_HIGH, preferred_element_type=jnp.float32)


def _proj_body(nf_ref, w_ref, o_ref):
    o_ref[...] = _dot(nf_ref[...], w_ref[...])


def _proj(nf, wpad):
    return pl.pallas_call(
        _proj_body,
        grid=(5,),
        in_specs=[pl.BlockSpec((2000, DF), lambda i: (i, 0)),
                  pl.BlockSpec((DF, 128), lambda i: (0, 0))],
        out_specs=pl.BlockSpec((2000, 128), lambda i: (i, 0)),
        out_shape=jax.ShapeDtypeStruct((N, 128), jnp.float32),
    )(nf, wpad)


def _sc_gather_body(p_hbm, is_hbm, id_hbm, gs_hbm, gd_hbm,
                    ivs, ivd, rvs, rvd, sems, semd):
    wid = lax.axis_index("s") * 2 + lax.axis_index("c")
    base = wid * PW

    @pl.loop(0, NCHUNK)
    def _(c):
        off = base + c * CHUNK
        pltpu.sync_copy(is_hbm.at[pl.ds(off, CHUNK)], ivs)
        pltpu.sync_copy(id_hbm.at[pl.ds(off, CHUNK)], ivd)
        cs = pltpu.async_copy(p_hbm.at[ivs], rvs, sems)
        cd = pltpu.async_copy(p_hbm.at[ivd], rvd, semd)
        cs.wait()
        cd.wait()
        pltpu.sync_copy(rvs, gs_hbm.at[pl.ds(off, CHUNK)])
        pltpu.sync_copy(rvd, gd_hbm.at[pl.ds(off, CHUNK)])


def _sc_gather(p8, idx_s, idx_d):
    mesh = plsc.VectorSubcoreMesh(core_axis_name="c", subcore_axis_name="s")
    f = pl.kernel(
        _sc_gather_body,
        mesh=mesh,
        compiler_params=pltpu.CompilerParams(use_tc_tiling_on_sc=False),
        out_type=[jax.ShapeDtypeStruct((EP, DE), jnp.float32),
                  jax.ShapeDtypeStruct((EP, DE), jnp.float32)],
        scratch_types=[pltpu.VMEM((CHUNK,), jnp.int32),
                       pltpu.VMEM((CHUNK,), jnp.int32),
                       pltpu.VMEM((CHUNK, DE), jnp.float32),
                       pltpu.VMEM((CHUNK, DE), jnp.float32),
                       pltpu.SemaphoreType.DMA,
                       pltpu.SemaphoreType.DMA],
    )
    return f(p8, idx_s, idx_d)


def _msg_body(ef_ref, et_ref, gs_ref, gd_ref, wbd_ref, be_ref, o_ref):
    x = ef_ref[...]
    et = et_ref[...]
    acc = gs_ref[...] + gd_ref[...]
    for t in range(TE):
        y = _dot(x, wbd_ref[t]) + be_ref[t]
        acc += jnp.where(et == t, y, 0.0)
    rows = (pl.program_id(0) * 2048
            + lax.broadcasted_iota(jnp.int32, (2048, 128), 0))
    o_ref[...] = jnp.where(rows < ROWS_REAL, jnp.maximum(acc, 0.0), 0.0)


def _msg(ef_rs, et_rep, gs_rs, gd_rs, wbd, be_rep):
    nb = ROWS // 2048
    return pl.pallas_call(
        _msg_body,
        grid=(nb,),
        in_specs=[pl.BlockSpec((2048, 128), lambda i: (i, 0)),
                  pl.BlockSpec((2048, 128), lambda i: (i, 0)),
                  pl.BlockSpec((2048, 128), lambda i: (i, 0)),
                  pl.BlockSpec((2048, 128), lambda i: (i, 0)),
                  pl.BlockSpec((TE, 128, 128), lambda i: (0, 0, 0)),
                  pl.BlockSpec((8, 128), lambda i: (0, 0))],
        out_specs=pl.BlockSpec((2048, 128), lambda i: (i, 0)),
        out_shape=jax.ShapeDtypeStruct((ROWS, 128), jnp.float32),
    )(ef_rs, et_rep, gs_rs, gd_rs, wbd, be_rep)


def _sc_scatter_body(val_hbm, didx_hbm, zero_hbm, part_hbm,
                     vv, iv, agg_sh, sem):
    cid = lax.axis_index("c")
    sid = lax.axis_index("s")

    @pl.when(sid == 0)
    def _():
        pltpu.sync_copy(zero_hbm, agg_sh)

    plsc.subcore_barrier()

    wid = sid * 2 + cid
    base = wid * PW

    @pl.loop(0, NCHUNK)
    def _(c):
        off = base + c * CHUNK
        pltpu.sync_copy(val_hbm.at[pl.ds(off, CHUNK)], vv)
        pltpu.sync_copy(didx_hbm.at[pl.ds(off, CHUNK)], iv)
        pltpu.sync_copy(vv, agg_sh.at[iv], add=True)

    plsc.subcore_barrier()
    rows = N // 16
    pltpu.sync_copy(agg_sh.at[pl.ds(sid * rows, rows)],
                    part_hbm.at[cid, pl.ds(sid * rows, rows)])


def _sc_scatter(vals, dst_idx, zeros_n):
    mesh = plsc.VectorSubcoreMesh(core_axis_name="c", subcore_axis_name="s")
    f = pl.kernel(
        _sc_scatter_body,
        mesh=mesh,
        compiler_params=pltpu.CompilerParams(use_tc_tiling_on_sc=False),
        out_type=jax.ShapeDtypeStruct((2, N, DE), jnp.float32),
        scratch_types=[pltpu.VMEM((CHUNK, DE), jnp.float32),
                       pltpu.VMEM((CHUNK,), jnp.int32),
                       pltpu.VMEM_SHARED((N, DE), jnp.float32),
                       pltpu.SemaphoreType.DMA],
    )
    return f(vals, dst_idx, zeros_n)


def _node_body(part_ref, nf_ref, nt_ref, wa_ref, wb_ref, bn_ref, o_ref):
    agg = part_ref[0] + part_ref[1]
    x = nf_ref[...]
    nt = nt_ref[...]
    y0 = jnp.maximum(_dot(agg, wa_ref[0]) + _dot(x, wb_ref[0]) + bn_ref[0], 0.0)
    y1 = jnp.maximum(_dot(agg, wa_ref[1]) + _dot(x, wb_ref[1]) + bn_ref[1], 0.0)
    o_ref[...] = jnp.where(nt == 0, y0, y1)


def _node(part, nf, ntype2, wa, wb, bn_pad):
    return pl.pallas_call(
        _node_body,
        grid=(5,),
        in_specs=[pl.BlockSpec((2, 2000, DE), lambda i: (0, i, 0)),
                  pl.BlockSpec((2000, DF), lambda i: (i, 0)),
                  pl.BlockSpec((2000, 1), lambda i: (i, 0)),
                  pl.BlockSpec((TN, DE, DF), lambda i: (0, 0, 0)),
                  pl.BlockSpec((TN, DF, DF), lambda i: (0, 0, 0)),
                  pl.BlockSpec((8, DF), lambda i: (0, 0))],
        out_specs=pl.BlockSpec((2000, DF), lambda i: (i, 0)),
        out_shape=jax.ShapeDtypeStruct((N, DF), jnp.float32),
    )(part, nf, ntype2, wa, wb, bn_pad)


def kernel(nf, ef, edge_index, etype, ntype, We, be, Wn, bn):
    src = edge_index[0]
    dst = edge_index[1]

    ws = jnp.transpose(We[:, DE:DE + DF, :], (1, 0, 2)).reshape(DF, TE * DE)
    wd = jnp.transpose(We[:, DE + DF:, :], (1, 0, 2)).reshape(DF, TE * DE)
    wpad = jnp.concatenate(
        [ws, wd, jnp.zeros((DF, 128 - 2 * TE * DE), jnp.float32)], axis=1)
    eye8 = jnp.eye(8, dtype=jnp.float32)
    wbd = jax.vmap(lambda w: jnp.kron(eye8, w))(We[:, :DE, :])
    be_rep = jnp.concatenate(
        [jnp.tile(be, (1, 8)), jnp.zeros((8 - TE, 128), jnp.float32)], axis=0)
    wa = Wn[:, :DE, :]
    wb = Wn[:, DE:, :]
    bn_pad = jnp.concatenate(
        [bn, jnp.zeros((8 - TN, DF), jnp.float32)], axis=0)

    pad = EP - E
    idx_s = jnp.pad(src * 8 + etype, (0, pad))
    idx_d = jnp.pad(dst * 8 + 3 + etype, (0, pad))
    dst_p = jnp.pad(dst, (0, pad))
    ef_rs = jnp.pad(ef, ((0, pad), (0, 0))).reshape(ROWS, 128)
    et_rep = jnp.repeat(jnp.pad(etype, (0, pad)), DE).reshape(ROWS, 128)
    ntype2 = ntype.reshape(N, 1)
    zeros_n = jnp.zeros((N, DE), jnp.float32)

    p = _proj(nf, wpad)
    p8 = p.reshape(N * 8, DE)

    gs, gd = _sc_gather(p8, idx_s, idx_d)
    gs_rs = gs.reshape(ROWS, 128)
    gd_rs = gd.reshape(ROWS, 128)

    msg_rs = _msg(ef_rs, et_rep, gs_rs, gd_rs, wbd, be_rep)
    msg = msg_rs.reshape(EP, DE)

    part = _sc_scatter(msg, dst_p, zeros_n)

    updated_nf = _node(part, nf, ntype2, wa, wb, bn_pad)

    return (updated_nf, msg[:E])
